# baseline (speedup 1.0000x reference)
import jax
import jax.numpy as jnp
from jax import lax
from jax.experimental import pallas as pl
from jax.experimental.pallas import tpu as pltpu
from jax.experimental.pallas import tpu_sc as plsc

_N = 10000
_E = 160000
_FEAT = 256
_D = 256
_R = 3
_NB = 1024
_GRID = 10
_NPAD = 10240
_NTILES = 16
_CHUNK = 64
_UNROLL = 4
_CPT = 160
_EPT = _CPT * _CHUNK
_EPAD = _NTILES * _EPT
_NACC = 10112
_ROWS_PT = _NACC // _NTILES
_SLABW = 128
_NSLAB = _FEAT // _SLABW


def _a_body(feat_ref, wall_ref, attn_ref, h12_ref, el_ref, er_ref):
    f = feat_ref[...]
    h = jnp.dot(f, wall_ref[...], preferred_element_type=jnp.float32)
    h12_ref[...] = jnp.swapaxes(h.reshape(_NB, _R * _NSLAB, _SLABW), 0, 1)
    els, ers = [], []
    for r in range(_R):
        hr = h[:, r * _D:(r + 1) * _D]
        al = attn_ref[2 * r][None, :]
        ar = attn_ref[2 * r + 1][None, :]
        els.append(jnp.sum(hr * al, axis=1).reshape(1, _NB))
        ers.append(jnp.sum(hr * ar, axis=1).reshape(1, _NB))
    el_ref[...] = jnp.concatenate(els, axis=0)
    er_ref[...] = jnp.concatenate(ers, axis=0)


def _phase_a(feat, w_all, attn):
    return pl.pallas_call(
        _a_body,
        grid=(_GRID,),
        in_specs=[
            pl.BlockSpec((_NB, _FEAT), lambda i: (i, 0)),
            pl.BlockSpec((_FEAT, _R * _D), lambda i: (0, 0)),
            pl.BlockSpec((2 * _R, _D), lambda i: (0, 0)),
        ],
        out_specs=[
            pl.BlockSpec((_R * _NSLAB, _NB, _SLABW), lambda i: (0, i, 0)),
            pl.BlockSpec((_R, _NB), lambda i: (0, i)),
            pl.BlockSpec((_R, _NB), lambda i: (0, i)),
        ],
        out_shape=[
            jax.ShapeDtypeStruct((_R * _NSLAB, _NPAD, _SLABW), jnp.float32),
            jax.ShapeDtypeStruct((_R, _NPAD), jnp.float32),
            jax.ShapeDtypeStruct((_R, _NPAD), jnp.float32),
        ],
    )(feat, w_all, attn)


def _sc_body(src_hbm, dst_hbm, el_hbm, er_hbm, h_hbm,
             acc_hbm, den_hbm,
             srcb_v, dstb_v, el_v, er_v, den_v, ex_v, rows_v,
             acc_sh, gsa, gsb, ssa, ssb, isem):
    c = lax.axis_index("c")
    s = lax.axis_index("s")

    zero16 = jnp.zeros((16,), jnp.float32)

    def zero_rows(i, carry):
        for k in range(_SLABW // 16):
            rows_v[0, i, pl.ds(k * 16, 16)] = zero16
        return carry

    def zero_den(i, carry):
        den_v[pl.ds(i * 16, 16)] = zero16
        return carry

    gsem = [gsa, gsb]
    ssem = [ssa, ssb]

    nbodies = _CPT // _UNROLL

    for r in range(_R):
        pltpu.sync_copy(el_hbm.at[pl.ds(r * _NPAD, _N)], el_v)
        pltpu.sync_copy(er_hbm.at[pl.ds(r * _NPAD, _N)], er_v)
        lax.fori_loop(0, _N // 16, zero_den, 0)
        slab = _NSLAB * r + c
        lax.fori_loop(0, _CHUNK, zero_rows, 0)
        for b in range(_ROWS_PT // _CHUNK):
            pltpu.sync_copy(
                rows_v.at[0],
                acc_sh.at[pl.ds(s * _ROWS_PT + b * _CHUNK, _CHUNK)])
        tail = _ROWS_PT % _CHUNK
        if tail:
            pltpu.sync_copy(
                rows_v.at[0, pl.ds(0, tail)],
                acc_sh.at[pl.ds(s * _ROWS_PT + _ROWS_PT - tail, tail)])

        base = slab * _NPAD
        tbase = (r * _NTILES + s) * _CPT * _CHUNK

        def fetch(p, k, cid, sync=False):
            off = tbase + cid * _CHUNK
            if sync:
                pltpu.sync_copy(src_hbm.at[pl.ds(off, _CHUNK)],
                                srcb_v.at[p, k])
                pltpu.sync_copy(dst_hbm.at[pl.ds(off, _CHUNK)],
                                dstb_v.at[p, k])
                return []
            return [
                pltpu.async_copy(src_hbm.at[pl.ds(off, _CHUNK)],
                                 srcb_v.at[p, k], isem),
                pltpu.async_copy(dst_hbm.at[pl.ds(off, _CHUNK)],
                                 dstb_v.at[p, k], isem),
            ]

        def exgen(p, k, cid):
            for g in range(_CHUNK // 16):
                sl = pl.ds(g * 16, 16)
                si = srcb_v[p, k, sl]
                di = dstb_v[p, k, sl]
                elg = plsc.load_gather(el_v, [si])
                erg = plsc.load_gather(er_v, [di])
                e = elg + erg
                e = jnp.maximum(e, 0.2 * e)
                ex = jnp.exp(e)
                eid = s * _EPT + cid * _CHUNK + g * 16 + lax.iota(jnp.int32, 16)
                ok = jnp.logical_and(eid < _E, cid < _CPT)
                ex = jnp.where(ok, ex, 0.0)
                ex_v[p, k, sl] = ex
                plsc.addupdate_scatter(den_v, [di], ex)
                srcb_v[p, k, sl] = si + base

        def scale(p, k):
            def grp(g, carry2):
                exg = ex_v[p, k, pl.ds(g * 16, 16)]
                for t in range(16):
                    w = jnp.broadcast_to(exg[t], (16,))
                    row = g * 16 + t
                    for cc in range(_SLABW // 16):
                        sl2 = pl.ds(cc * 16, 16)
                        rows_v[k % 2, row, sl2] = rows_v[k % 2, row, sl2] * w
                return carry2
            lax.fori_loop(0, _CHUNK // 16, grp, 0)

        def gath(p, k):
            return pltpu.async_copy(
                h_hbm.at[srcb_v.at[p, k]], rows_v.at[k % 2], gsem[k % 2])

        def scat(p, k):
            return pltpu.async_copy(
                rows_v.at[k % 2], acc_sh.at[dstb_v.at[p, k]], ssem[k % 2],
                add=True)

        for k in range(_UNROLL):
            fetch(0, k, k, sync=True)
        plsc.subcore_barrier()
        for k in range(_UNROLL):
            exgen(0, k, k)

        def body(j, carry):
            p = j % 2
            pn = 1 - p
            g0 = gath(p, 0)
            g1 = gath(p, 1)
            fds = []
            for k in range(_UNROLL):
                fds += fetch(pn, k, (j + 1) * _UNROLL + k)
            for fd in fds:
                fd.wait()
            exgen(pn, 0, (j + 1) * _UNROLL + 0)
            exgen(pn, 1, (j + 1) * _UNROLL + 1)
            g0.wait()
            scale(p, 0)
            sc0 = scat(p, 0)
            g1.wait()
            scale(p, 1)
            sc1 = scat(p, 1)
            sc0.wait()
            g2 = gath(p, 2)
            exgen(pn, 2, (j + 1) * _UNROLL + 2)
            g2.wait()
            scale(p, 2)
            sc2 = scat(p, 2)
            sc1.wait()
            g3 = gath(p, 3)
            exgen(pn, 3, (j + 1) * _UNROLL + 3)
            g3.wait()
            scale(p, 3)
            sc3 = scat(p, 3)
            sc2.wait()
            sc3.wait()
            return carry

        lax.fori_loop(0, nbodies, body, 0)
        plsc.subcore_barrier()
        pltpu.sync_copy(
            acc_sh.at[pl.ds(s * _ROWS_PT, _ROWS_PT)],
            acc_hbm.at[slab, pl.ds(s * _ROWS_PT, _ROWS_PT)])
        dslot = (r * 2 * _NTILES + c * _NTILES + s) * _N
        pltpu.sync_copy(den_v, den_hbm.at[pl.ds(dslot, _N)])


def _phase_b(src_flat, dst_flat, el, er, h_flat):
    mesh = plsc.VectorSubcoreMesh(core_axis_name="c", subcore_axis_name="s")
    f = pl.kernel(
        _sc_body,
        out_type=[
            jax.ShapeDtypeStruct((_R * _NSLAB, _NACC, _SLABW), jnp.float32),
            jax.ShapeDtypeStruct((_R * 2 * _NTILES * _N,), jnp.float32),
        ],
        mesh=mesh,
        compiler_params=pltpu.CompilerParams(needs_layout_passes=False),
        scratch_types=[
            pltpu.VMEM((2, _UNROLL, _CHUNK), jnp.int32),
            pltpu.VMEM((2, _UNROLL, _CHUNK), jnp.int32),
            pltpu.VMEM((_N,), jnp.float32),
            pltpu.VMEM((_N,), jnp.float32),
            pltpu.VMEM((_N,), jnp.float32),
            pltpu.VMEM((2, _UNROLL, _CHUNK), jnp.float32),
            pltpu.VMEM((2, _CHUNK, _SLABW), jnp.float32),
            pltpu.VMEM_SHARED((_NACC, _SLABW), jnp.float32),
            pltpu.SemaphoreType.DMA,
            pltpu.SemaphoreType.DMA,
            pltpu.SemaphoreType.DMA,
            pltpu.SemaphoreType.DMA,
            pltpu.SemaphoreType.DMA,
        ],
    )
    return f(src_flat, dst_flat, el, er, h_flat)


def _c1_body(acc_ref, den_ref, bias_ref, w1_ref, b1_ref, q_ref, z_ref, wp_ref):
    i = pl.program_id(0)
    rowid = i * _NB + lax.broadcasted_iota(jnp.int32, (_NB, 1), 0)
    valid = rowid < _N
    dsum = jnp.sum(den_ref[...], axis=1)
    wps = []
    for r in range(_R):
        d = dsum[r].reshape(_NB, 1)
        d = jnp.where(d == 0.0, 1.0, d)
        z = jnp.concatenate([acc_ref[_NSLAB * r + k] for k in range(_NSLAB)],
                            axis=1) / d
        z = z + bias_ref[r][None, :]
        z = jnp.where(z > 0, z, jnp.exp(jnp.minimum(z, 0.0)) - 1.0)
        z_ref[r] = z
        p = jnp.tanh(jnp.dot(z, w1_ref[...], preferred_element_type=jnp.float32)
                     + b1_ref[...])
        p = jnp.dot(p, q_ref[...], preferred_element_type=jnp.float32)
        p = jnp.where(valid, p, 0.0)
        wps.append(jnp.sum(p).reshape(1, 1))
    wps.append(jnp.zeros((1, 128 - _R), jnp.float32))
    wp_ref[...] = jnp.concatenate(wps, axis=1).reshape(1, 1, 128)


def _phase_c1(acc, den, bias_st, w_sem1, b_sem1, q_sem):
    return pl.pallas_call(
        _c1_body,
        grid=(_GRID,),
        in_specs=[
            pl.BlockSpec((_R * _NSLAB, _NB, _SLABW), lambda i: (0, i, 0)),
            pl.BlockSpec((_R, _NTILES, _NB), lambda i: (0, 0, i)),
            pl.BlockSpec((_R, _D), lambda i: (0, 0)),
            pl.BlockSpec((_D, 128), lambda i: (0, 0)),
            pl.BlockSpec((1, 128), lambda i: (0, 0)),
            pl.BlockSpec((128, 1), lambda i: (0, 0)),
        ],
        out_specs=[
            pl.BlockSpec((_R, _NB, _D), lambda i: (0, i, 0)),
            pl.BlockSpec((1, 1, 128), lambda i: (i, 0, 0)),
        ],
        out_shape=[
            jax.ShapeDtypeStruct((_R, _NPAD, _D), jnp.float32),
            jax.ShapeDtypeStruct((_GRID, 1, 128), jnp.float32),
        ],
    )(acc, den, bias_st, w_sem1, b_sem1, q_sem)


def _c2_body(z_ref, wp_ref, out_ref):
    w = jnp.sum(wp_ref[...], axis=0)
    w3 = w[:, :_R] / float(_N)
    m = jnp.max(w3, axis=1, keepdims=True)
    e = jnp.exp(w3 - m)
    beta = e / jnp.sum(e, axis=1, keepdims=True)
    out_ref[...] = (beta[0, 0] * z_ref[0] + beta[0, 1] * z_ref[1]
                    + beta[0, 2] * z_ref[2])


def _phase_c2(z, wpart):
    return pl.pallas_call(
        _c2_body,
        grid=(_GRID,),
        in_specs=[
            pl.BlockSpec((_R, _NB, _D), lambda i: (0, i, 0)),
            pl.BlockSpec((_GRID, 1, 128), lambda i: (0, 0, 0)),
        ],
        out_specs=pl.BlockSpec((_NB, _D), lambda i: (i, 0)),
        out_shape=jax.ShapeDtypeStruct((_N, _D), jnp.float32),
    )(z, wpart)


def kernel(feat, edge_index_r0, edge_index_r1, edge_index_r2,
           W_fc_r0, attn_l_r0, attn_r_r0, bias_r0,
           W_fc_r1, attn_l_r1, attn_r_r1, bias_r1,
           W_fc_r2, attn_l_r2, attn_r_r2, bias_r2,
           W_sem1, b_sem1, q_sem):
    w_all = jnp.concatenate([W_fc_r0, W_fc_r1, W_fc_r2], axis=1)
    attn = jnp.concatenate([attn_l_r0, attn_r_r0, attn_l_r1, attn_r_r1,
                            attn_l_r2, attn_r_r2], axis=0)
    bias_st = jnp.stack([bias_r0, bias_r1, bias_r2], axis=0)

    srcs, dsts = [], []
    for ei in (edge_index_r0, edge_index_r1, edge_index_r2):
        ei = ei.astype(jnp.int32)
        pad = jnp.zeros((2, _EPAD - _E), jnp.int32)
        ep = jnp.concatenate([ei, pad], axis=1)
        srcs.append(ep[0])
        dsts.append(ep[1])
    tailpad = jnp.zeros((_UNROLL * _CHUNK,), jnp.int32)
    src_flat = jnp.concatenate(srcs + [tailpad], axis=0)
    dst_flat = jnp.concatenate(dsts + [tailpad], axis=0)

    h12, el, er = _phase_a(feat, w_all, attn)
    h_flat = h12.reshape(_R * _NSLAB * _NPAD, _SLABW)
    acc, den_flat = _phase_b(src_flat, dst_flat, el.reshape(_R * _NPAD),
                             er.reshape(_R * _NPAD), h_flat)
    den = den_flat.reshape(_R, 2 * _NTILES, _N)
    z, wpart = _phase_c1(acc, den, bias_st, W_sem1,
                         b_sem1.reshape(1, 128), q_sem)
    return _phase_c2(z, wpart)


# speedup vs baseline: 1.0004x; 1.0004x over previous
import jax
import jax.numpy as jnp
from jax import lax
from jax.experimental import pallas as pl
from jax.experimental.pallas import tpu as pltpu
from jax.experimental.pallas import tpu_sc as plsc

_N = 10000
_E = 160000
_FEAT = 256
_D = 256
_R = 3
_NB = 1024
_GRID = 10
_NPAD = 10240
_NTILES = 16
_CHUNK = 64
_UNROLL = 4
_CPT = 160
_EPT = _CPT * _CHUNK
_EPAD = _NTILES * _EPT
_NACC = 10112
_ROWS_PT = _NACC // _NTILES
_SLABW = 128
_NSLAB = _FEAT // _SLABW


def _a_body(feat_ref, wall_ref, attn_ref, h12_ref, el_ref, er_ref):
    f = feat_ref[...]
    h = jnp.dot(f, wall_ref[...], preferred_element_type=jnp.float32)
    h12_ref[...] = jnp.swapaxes(h.reshape(_NB, _R * _NSLAB, _SLABW), 0, 1)
    els, ers = [], []
    for r in range(_R):
        hr = h[:, r * _D:(r + 1) * _D]
        al = attn_ref[2 * r][None, :]
        ar = attn_ref[2 * r + 1][None, :]
        els.append(jnp.sum(hr * al, axis=1).reshape(1, _NB))
        ers.append(jnp.sum(hr * ar, axis=1).reshape(1, _NB))
    el_ref[...] = jnp.concatenate(els, axis=0)
    er_ref[...] = jnp.concatenate(ers, axis=0)


def _phase_a(feat, w_all, attn):
    return pl.pallas_call(
        _a_body,
        grid=(_GRID,),
        in_specs=[
            pl.BlockSpec((_NB, _FEAT), lambda i: (i, 0)),
            pl.BlockSpec((_FEAT, _R * _D), lambda i: (0, 0)),
            pl.BlockSpec((2 * _R, _D), lambda i: (0, 0)),
        ],
        out_specs=[
            pl.BlockSpec((_R * _NSLAB, _NB, _SLABW), lambda i: (0, i, 0)),
            pl.BlockSpec((_R, _NB), lambda i: (0, i)),
            pl.BlockSpec((_R, _NB), lambda i: (0, i)),
        ],
        out_shape=[
            jax.ShapeDtypeStruct((_R * _NSLAB, _NPAD, _SLABW), jnp.float32),
            jax.ShapeDtypeStruct((_R, _NPAD), jnp.float32),
            jax.ShapeDtypeStruct((_R, _NPAD), jnp.float32),
        ],
    )(feat, w_all, attn)


def _sc_body(src_hbm, dst_hbm, el_hbm, er_hbm, h_hbm,
             acc_hbm, den_hbm,
             srcb_v, dstb_v, el_v, er_v, den_v, ex_v, rows_v,
             acc_sh, gsa, gsb, ssa, ssb, isem):
    c = lax.axis_index("c")
    s = lax.axis_index("s")

    zero16 = jnp.zeros((16,), jnp.float32)

    def zero_rows(i, carry):
        for k in range(_SLABW // 16):
            rows_v[0, i, pl.ds(k * 16, 16)] = zero16
        return carry

    def zero_den(i, carry):
        den_v[pl.ds(i * 16, 16)] = zero16
        return carry

    gsem = [gsa, gsb]
    ssem = [ssa, ssb]

    nbodies = _CPT // _UNROLL

    for r in range(_R):
        pltpu.sync_copy(el_hbm.at[pl.ds(r * _NPAD, _N)], el_v)
        pltpu.sync_copy(er_hbm.at[pl.ds(r * _NPAD, _N)], er_v)
        lax.fori_loop(0, _N // 16, zero_den, 0)
        slab = _NSLAB * r + c
        lax.fori_loop(0, _CHUNK, zero_rows, 0)
        for b in range(_ROWS_PT // _CHUNK):
            pltpu.sync_copy(
                rows_v.at[0],
                acc_sh.at[pl.ds(s * _ROWS_PT + b * _CHUNK, _CHUNK)])
        tail = _ROWS_PT % _CHUNK
        if tail:
            pltpu.sync_copy(
                rows_v.at[0, pl.ds(0, tail)],
                acc_sh.at[pl.ds(s * _ROWS_PT + _ROWS_PT - tail, tail)])

        base = slab * _NPAD
        tbase = (r * _NTILES + s) * _CPT * _CHUNK

        def fetch(p, k, cid, sync=False):
            off = tbase + cid * _CHUNK
            if sync:
                pltpu.sync_copy(src_hbm.at[pl.ds(off, _CHUNK)],
                                srcb_v.at[p, k])
                pltpu.sync_copy(dst_hbm.at[pl.ds(off, _CHUNK)],
                                dstb_v.at[p, k])
                return []
            return [
                pltpu.async_copy(src_hbm.at[pl.ds(off, _CHUNK)],
                                 srcb_v.at[p, k], isem),
                pltpu.async_copy(dst_hbm.at[pl.ds(off, _CHUNK)],
                                 dstb_v.at[p, k], isem),
            ]

        def exgen(p, k, cid):
            for g in range(_CHUNK // 16):
                sl = pl.ds(g * 16, 16)
                si = srcb_v[p, k, sl]
                di = dstb_v[p, k, sl]
                elg = plsc.load_gather(el_v, [si])
                erg = plsc.load_gather(er_v, [di])
                e = elg + erg
                e = jnp.maximum(e, 0.2 * e)
                ex = jnp.exp(e)
                eid = s * _EPT + cid * _CHUNK + g * 16 + lax.iota(jnp.int32, 16)
                ok = jnp.logical_and(eid < _E, cid < _CPT)
                ex = jnp.where(ok, ex, 0.0)
                ex_v[p, k, sl] = ex
                plsc.addupdate_scatter(den_v, [di], ex)
                srcb_v[p, k, sl] = si + base

        def scale(p, k):
            def grp(g, carry2):
                exg = ex_v[p, k, pl.ds(g * 16, 16)]
                for t in range(16):
                    w = jnp.broadcast_to(exg[t], (16,))
                    row = g * 16 + t
                    for cc in range(_SLABW // 16):
                        sl2 = pl.ds(cc * 16, 16)
                        rows_v[k % 2, row, sl2] = rows_v[k % 2, row, sl2] * w
                return carry2
            lax.fori_loop(0, _CHUNK // 16, grp, 0)

        def gath(p, k):
            return pltpu.async_copy(
                h_hbm.at[srcb_v.at[p, k]], rows_v.at[k % 2], gsem[k % 2])

        def scat(p, k):
            return pltpu.async_copy(
                rows_v.at[k % 2], acc_sh.at[dstb_v.at[p, k]], ssem[k % 2],
                add=True)

        for k in range(_UNROLL):
            fetch(0, k, k, sync=True)
        plsc.subcore_barrier()
        for k in range(_UNROLL):
            exgen(0, k, k)

        def body(j, carry):
            p = j % 2
            pn = 1 - p
            fds = []
            for k in range(_UNROLL):
                fds += fetch(pn, k, (j + 1) * _UNROLL + k)
            g0 = gath(p, 0)
            g1 = gath(p, 1)
            for fd in fds:
                fd.wait()
            exgen(pn, 0, (j + 1) * _UNROLL + 0)
            exgen(pn, 1, (j + 1) * _UNROLL + 1)
            g0.wait()
            scale(p, 0)
            sc0 = scat(p, 0)
            g1.wait()
            scale(p, 1)
            sc1 = scat(p, 1)
            sc0.wait()
            g2 = gath(p, 2)
            exgen(pn, 2, (j + 1) * _UNROLL + 2)
            g2.wait()
            scale(p, 2)
            sc2 = scat(p, 2)
            sc1.wait()
            g3 = gath(p, 3)
            exgen(pn, 3, (j + 1) * _UNROLL + 3)
            g3.wait()
            scale(p, 3)
            sc3 = scat(p, 3)
            sc2.wait()
            sc3.wait()
            return carry

        lax.fori_loop(0, nbodies, body, 0)
        plsc.subcore_barrier()
        pltpu.sync_copy(
            acc_sh.at[pl.ds(s * _ROWS_PT, _ROWS_PT)],
            acc_hbm.at[slab, pl.ds(s * _ROWS_PT, _ROWS_PT)])
        dslot = (r * 2 * _NTILES + c * _NTILES + s) * _N
        pltpu.sync_copy(den_v, den_hbm.at[pl.ds(dslot, _N)])


def _phase_b(src_flat, dst_flat, el, er, h_flat):
    mesh = plsc.VectorSubcoreMesh(core_axis_name="c", subcore_axis_name="s")
    f = pl.kernel(
        _sc_body,
        out_type=[
            jax.ShapeDtypeStruct((_R * _NSLAB, _NACC, _SLABW), jnp.float32),
            jax.ShapeDtypeStruct((_R * 2 * _NTILES * _N,), jnp.float32),
        ],
        mesh=mesh,
        compiler_params=pltpu.CompilerParams(needs_layout_passes=False),
        scratch_types=[
            pltpu.VMEM((2, _UNROLL, _CHUNK), jnp.int32),
            pltpu.VMEM((2, _UNROLL, _CHUNK), jnp.int32),
            pltpu.VMEM((_N,), jnp.float32),
            pltpu.VMEM((_N,), jnp.float32),
            pltpu.VMEM((_N,), jnp.float32),
            pltpu.VMEM((2, _UNROLL, _CHUNK), jnp.float32),
            pltpu.VMEM((2, _CHUNK, _SLABW), jnp.float32),
            pltpu.VMEM_SHARED((_NACC, _SLABW), jnp.float32),
            pltpu.SemaphoreType.DMA,
            pltpu.SemaphoreType.DMA,
            pltpu.SemaphoreType.DMA,
            pltpu.SemaphoreType.DMA,
            pltpu.SemaphoreType.DMA,
        ],
    )
    return f(src_flat, dst_flat, el, er, h_flat)


def _c1_body(acc_ref, den_ref, bias_ref, w1_ref, b1_ref, q_ref, z_ref, wp_ref):
    i = pl.program_id(0)
    rowid = i * _NB + lax.broadcasted_iota(jnp.int32, (_NB, 1), 0)
    valid = rowid < _N
    dsum = jnp.sum(den_ref[...], axis=1)
    wps = []
    for r in range(_R):
        d = dsum[r].reshape(_NB, 1)
        d = jnp.where(d == 0.0, 1.0, d)
        z = jnp.concatenate([acc_ref[_NSLAB * r + k] for k in range(_NSLAB)],
                            axis=1) / d
        z = z + bias_ref[r][None, :]
        z = jnp.where(z > 0, z, jnp.exp(jnp.minimum(z, 0.0)) - 1.0)
        z_ref[r] = z
        p = jnp.tanh(jnp.dot(z, w1_ref[...], preferred_element_type=jnp.float32)
                     + b1_ref[...])
        p = jnp.dot(p, q_ref[...], preferred_element_type=jnp.float32)
        p = jnp.where(valid, p, 0.0)
        wps.append(jnp.sum(p).reshape(1, 1))
    wps.append(jnp.zeros((1, 128 - _R), jnp.float32))
    wp_ref[...] = jnp.concatenate(wps, axis=1).reshape(1, 1, 128)


def _phase_c1(acc, den, bias_st, w_sem1, b_sem1, q_sem):
    return pl.pallas_call(
        _c1_body,
        grid=(_GRID,),
        in_specs=[
            pl.BlockSpec((_R * _NSLAB, _NB, _SLABW), lambda i: (0, i, 0)),
            pl.BlockSpec((_R, _NTILES, _NB), lambda i: (0, 0, i)),
            pl.BlockSpec((_R, _D), lambda i: (0, 0)),
            pl.BlockSpec((_D, 128), lambda i: (0, 0)),
            pl.BlockSpec((1, 128), lambda i: (0, 0)),
            pl.BlockSpec((128, 1), lambda i: (0, 0)),
        ],
        out_specs=[
            pl.BlockSpec((_R, _NB, _D), lambda i: (0, i, 0)),
            pl.BlockSpec((1, 1, 128), lambda i: (i, 0, 0)),
        ],
        out_shape=[
            jax.ShapeDtypeStruct((_R, _NPAD, _D), jnp.float32),
            jax.ShapeDtypeStruct((_GRID, 1, 128), jnp.float32),
        ],
    )(acc, den, bias_st, w_sem1, b_sem1, q_sem)


def _c2_body(z_ref, wp_ref, out_ref):
    w = jnp.sum(wp_ref[...], axis=0)
    w3 = w[:, :_R] / float(_N)
    m = jnp.max(w3, axis=1, keepdims=True)
    e = jnp.exp(w3 - m)
    beta = e / jnp.sum(e, axis=1, keepdims=True)
    out_ref[...] = (beta[0, 0] * z_ref[0] + beta[0, 1] * z_ref[1]
                    + beta[0, 2] * z_ref[2])


def _phase_c2(z, wpart):
    return pl.pallas_call(
        _c2_body,
        grid=(_GRID,),
        in_specs=[
            pl.BlockSpec((_R, _NB, _D), lambda i: (0, i, 0)),
            pl.BlockSpec((_GRID, 1, 128), lambda i: (0, 0, 0)),
        ],
        out_specs=pl.BlockSpec((_NB, _D), lambda i: (i, 0)),
        out_shape=jax.ShapeDtypeStruct((_N, _D), jnp.float32),
    )(z, wpart)


def kernel(feat, edge_index_r0, edge_index_r1, edge_index_r2,
           W_fc_r0, attn_l_r0, attn_r_r0, bias_r0,
           W_fc_r1, attn_l_r1, attn_r_r1, bias_r1,
           W_fc_r2, attn_l_r2, attn_r_r2, bias_r2,
           W_sem1, b_sem1, q_sem):
    w_all = jnp.concatenate([W_fc_r0, W_fc_r1, W_fc_r2], axis=1)
    attn = jnp.concatenate([attn_l_r0, attn_r_r0, attn_l_r1, attn_r_r1,
                            attn_l_r2, attn_r_r2], axis=0)
    bias_st = jnp.stack([bias_r0, bias_r1, bias_r2], axis=0)

    srcs, dsts = [], []
    for ei in (edge_index_r0, edge_index_r1, edge_index_r2):
        ei = ei.astype(jnp.int32)
        pad = jnp.zeros((2, _EPAD - _E), jnp.int32)
        ep = jnp.concatenate([ei, pad], axis=1)
        srcs.append(ep[0])
        dsts.append(ep[1])
    tailpad = jnp.zeros((_UNROLL * _CHUNK,), jnp.int32)
    src_flat = jnp.concatenate(srcs + [tailpad], axis=0)
    dst_flat = jnp.concatenate(dsts + [tailpad], axis=0)

    h12, el, er = _phase_a(feat, w_all, attn)
    h_flat = h12.reshape(_R * _NSLAB * _NPAD, _SLABW)
    acc, den_flat = _phase_b(src_flat, dst_flat, el.reshape(_R * _NPAD),
                             er.reshape(_R * _NPAD), h_flat)
    den = den_flat.reshape(_R, 2 * _NTILES, _N)
    z, wpart = _phase_c1(acc, den, bias_st, W_sem1,
                         b_sem1.reshape(1, 128), q_sem)
    return _phase_c2(z, wpart)
